# baseline (speedup 1.0000x reference)
import functools

import jax
import jax.numpy as jnp
from jax import lax
from jax.experimental import pallas as pl
from jax.experimental.pallas import tpu as pltpu

_F32 = jnp.float32
_I32 = jnp.int32

_D = 2048
_LM = 128
_MAX_LM = 64
_B = 4
_K = 64
_NTOK = _B * 4096
_ROWS = _B * _K
_RPS = 8
_NSTEP = _ROWS // _RPS


def _tc_body(sgr_ref, misc_ref,
             ttt_ref, sgrv_ref, tp_ref, ema_ref, w_ref, aein_ref, aiin_ref,
             *rest):
    xrefs = rest[:_RPS]
    aeout_ref, aiout_ref, lmout_ref, scalout_ref = rest[_RPS:_RPS + 4]
    coef_ref, acc_ref, oh_ref = rest[_RPS + 4:]
    s = pl.program_id(0)

    @pl.when(s == 0)
    def _():
        for b in range(_B):
            iota0 = lax.broadcasted_iota(_I32, (4096, _K), 0)
            srow = sgrv_ref[b:b + 1, :]
            oh_ref[...] = (iota0 == srow).astype(_F32)
            imp = jax.lax.dot_general(
                ttt_ref[b:b + 1, :], oh_ref[...],
                (((1,), (0,)), ((), ())), preferred_element_type=_F32)
            mval = jnp.max(imp)
            e = jnp.exp(_F32(5.0) * (imp - mval))
            den = jnp.sum(e)
            c = e * (_F32(0.25) / den)
            coef_ref[b:b + 1, :] = jnp.concatenate(
                [c, jnp.zeros((1, _K), _F32)], axis=1)
        acc_ref[...] = jnp.zeros((1, _D), _F32)

    base = s * _RPS
    lane = lax.broadcasted_iota(_I32, (1, _LM), 1)
    coefrow = coef_ref[pl.ds(s // (_K // _RPS), 1), :]
    lane0 = base % _K
    acc = acc_ref[...]
    for k in range(_RPS):
        cjk = jnp.sum(jnp.where(lane == lane0 + k, coefrow, _F32(0.0)))
        acc = acc + cjk * xrefs[k][0]
    acc_ref[...] = acc

    @pl.when(s == _NSTEP - 1)
    def _():
        raw = acc_ref[...]
        lm = jax.lax.dot_general(
            raw, w_ref[...], (((1,), (1,)), ((), ())),
            preferred_element_type=_F32)
        lmout_ref[...] = lm

        mean_err = jnp.sum(ttt_ref[...]) * _F32(1.0 / _NTOK)
        full_prob = jnp.sum(tp_ref[...]) * _F32(0.25)
        emav = ema_ref[...]
        lane = lax.broadcasted_iota(_I32, (1, _LM), 1)
        m = jnp.sum(jnp.where(lane == 0, emav, _F32(0.0)))
        v = jnp.sum(jnp.where(lane == 1, emav, _F32(0.0)))
        delta = mean_err - m
        new_mean = m + _F32(0.05) * delta
        new_var = v * _F32(0.95) + _F32(0.05) * delta * delta
        thr = jnp.maximum(
            new_mean + _F32(0.5) * jnp.sqrt(jnp.maximum(new_var, _F32(1e-8))),
            _F32(0.3))
        skip = jnp.logical_and(mean_err < thr, full_prob < _F32(0.5))
        af = jnp.where(skip, _F32(0.0), _F32(1.0))
        score = mean_err * full_prob + _F32(1e-6)

        n = misc_ref[0]
        aeout_ref[...] = aein_ref[...]
        old = aein_ref[pl.ds(n, 1), :]
        aeout_ref[pl.ds(n, 1), :] = af * lm + (_F32(1.0) - af) * old

        lane64 = lax.broadcasted_iota(_I32, (1, _MAX_LM), 1)
        ai = aiin_ref[...]
        newi = af * score + (_F32(1.0) - af) * ai
        aiout_ref[...] = jnp.where(lane64 == n, newi, ai)

        sv = jnp.where(lane == 0, new_mean,
                       jnp.where(lane == 1, new_var,
                                 jnp.where(lane == 2, af, _F32(0.0))))
        scalout_ref[...] = sv


def kernel(scan_out, ttt_importance, tier_probs, sgr_indices, W_compress,
           archived_embeddings, archived_importance, err_ema_mean, err_ema_var,
           n_archived):
    x3 = scan_out.reshape(_NTOK, 1, _D)
    sgrv = sgr_indices.astype(_I32)
    base = (jnp.arange(_B, dtype=_I32) * 4096)[:, None]
    sgr = (sgrv + base).reshape(_ROWS)
    tp = jnp.zeros((1, _LM), _F32).at[0, :4].set(tier_probs[:, 2].astype(_F32))
    ema = jnp.zeros((1, _LM), _F32).at[0, 0].set(err_ema_mean).at[0, 1].set(err_ema_var)
    misc = jnp.zeros((8,), _I32).at[0].set(jnp.asarray(n_archived, _I32))
    aiin = archived_importance.reshape(1, _MAX_LM)

    const = lambda i, p1, p2: (0, 0)
    in_specs = [
        pl.BlockSpec((_B, 4096), const),
        pl.BlockSpec((_B, _K), const),
        pl.BlockSpec((1, _LM), const),
        pl.BlockSpec((1, _LM), const),
        pl.BlockSpec((_LM, _D), const),
        pl.BlockSpec((_MAX_LM, _LM), const),
        pl.BlockSpec((1, _MAX_LM), const),
    ] + [
        pl.BlockSpec((1, 1, _D),
                     (lambda i, p1, p2, k=k: (p1[_RPS * i + k], 0, 0)))
        for k in range(_RPS)
    ]
    out_specs = [
        pl.BlockSpec((_MAX_LM, _LM), const),
        pl.BlockSpec((1, _MAX_LM), const),
        pl.BlockSpec((1, _LM), const),
        pl.BlockSpec((1, _LM), const),
    ]
    grid_spec = pltpu.PrefetchScalarGridSpec(
        num_scalar_prefetch=2,
        grid=(_NSTEP,),
        in_specs=in_specs,
        out_specs=out_specs,
        scratch_shapes=[
            pltpu.VMEM((_B, _LM), _F32),
            pltpu.VMEM((1, _D), _F32),
            pltpu.VMEM((4096, _K), _F32),
        ],
    )
    aeout, aiout, lmout, scal = pl.pallas_call(
        _tc_body,
        grid_spec=grid_spec,
        out_shape=[
            jax.ShapeDtypeStruct((_MAX_LM, _LM), _F32),
            jax.ShapeDtypeStruct((1, _MAX_LM), _F32),
            jax.ShapeDtypeStruct((1, _LM), _F32),
            jax.ShapeDtypeStruct((1, _LM), _F32),
        ],
    )(sgr, misc, ttt_importance, sgrv, tp, ema, W_compress,
      archived_embeddings, aiin, *([x3] * _RPS))

    return (aeout, aiout.reshape(_MAX_LM), lmout.reshape(_LM),
            scal[0, 2] > 0.5, scal[0, 0], scal[0, 1])
